# baseline (speedup 1.0000x reference)
import jax
import jax.numpy as jnp
from jax.experimental import pallas as pl

_ROWS = 128
_COLS = 100000
_BLOCK_ROWS = 8

_ROTS = ((13, 15, 26, 6), (17, 29, 16, 24))


def _bern_kernel(p_ref, o_ref):
    r = pl.program_id(0)
    R, C = p_ref.shape
    row = jax.lax.broadcasted_iota(jnp.uint32, (R, C), 0)
    col = jax.lax.broadcasted_iota(jnp.uint32, (R, C), 1)
    base = (r * R).astype(jnp.uint32)
    idx = (base + row) * jnp.uint32(C) + col

    k0 = jnp.uint32(0)
    k1 = jnp.uint32(42)
    ks = (k0, k1, k0 ^ k1 ^ jnp.uint32(0x1BD11BDA))
    x0 = jnp.full_like(idx, k0)
    x1 = idx + k1
    for i in range(5):
        for rot in _ROTS[i % 2]:
            x0 = x0 + x1
            x1 = (x1 << rot) | (x1 >> (32 - rot))
            x1 = x1 ^ x0
        x0 = x0 + ks[(i + 1) % 3]
        x1 = x1 + ks[(i + 2) % 3] + jnp.uint32(i + 1)

    bits = x0 ^ x1
    fb = (bits >> jnp.uint32(9)) | jnp.uint32(0x3F800000)
    u = jax.lax.bitcast_convert_type(fb, jnp.float32) - jnp.float32(1.0)
    o_ref[...] = (u < p_ref[...]).astype(jnp.float32)


def kernel(probs):
    return pl.pallas_call(
        _bern_kernel,
        grid=(_ROWS // _BLOCK_ROWS,),
        in_specs=[pl.BlockSpec((_BLOCK_ROWS, _COLS), lambda r: (r, 0))],
        out_specs=pl.BlockSpec((_BLOCK_ROWS, _COLS), lambda r: (r, 0)),
        out_shape=jax.ShapeDtypeStruct((_ROWS, _COLS), probs.dtype),
    )(probs)


# speedup vs baseline: 1.0002x; 1.0002x over previous
import jax
import jax.numpy as jnp
from jax.experimental import pallas as pl
from jax.experimental.pallas import tpu as pltpu

_ROWS = 128
_COLS = 100000
_BLOCK_ROWS = 8

_ROTS = ((13, 15, 26, 6), (17, 29, 16, 24))


def _bern_kernel(p_ref, o_ref):
    r = pl.program_id(0)
    R, C = p_ref.shape
    row = jax.lax.broadcasted_iota(jnp.uint32, (R, C), 0)
    col = jax.lax.broadcasted_iota(jnp.uint32, (R, C), 1)
    base = (r * R).astype(jnp.uint32)
    idx = (base + row) * jnp.uint32(C) + col

    k0 = jnp.uint32(0)
    k1 = jnp.uint32(42)
    ks = (k0, k1, k0 ^ k1 ^ jnp.uint32(0x1BD11BDA))
    x0 = jnp.full_like(idx, k0)
    x1 = idx + k1
    for i in range(5):
        for rot in _ROTS[i % 2]:
            x0 = x0 + x1
            x1 = (x1 << rot) | (x1 >> (32 - rot))
            x1 = x1 ^ x0
        x0 = x0 + ks[(i + 1) % 3]
        x1 = x1 + ks[(i + 2) % 3] + jnp.uint32(i + 1)

    bits = x0 ^ x1
    fb = (bits >> jnp.uint32(9)) | jnp.uint32(0x3F800000)
    u = jax.lax.bitcast_convert_type(fb, jnp.float32) - jnp.float32(1.0)
    o_ref[...] = (u < p_ref[...]).astype(jnp.float32)


def kernel(probs):
    return pl.pallas_call(
        _bern_kernel,
        grid=(_ROWS // _BLOCK_ROWS,),
        in_specs=[pl.BlockSpec((_BLOCK_ROWS, _COLS), lambda r: (r, 0))],
        out_specs=pl.BlockSpec((_BLOCK_ROWS, _COLS), lambda r: (r, 0)),
        out_shape=jax.ShapeDtypeStruct((_ROWS, _COLS), probs.dtype),
        compiler_params=pltpu.CompilerParams(
            dimension_semantics=("parallel",)),
    )(probs)
